# baseline (speedup 1.0000x reference)
import functools
import math

import jax
import jax.numpy as jnp
from jax import lax
from jax.experimental import pallas as pl
from jax.experimental.pallas import tpu as pltpu
from jax.experimental.pallas import tpu_sc as plsc

B, S, H, SKV, G, D, T, K = 32, 1, 16, 8192, 1, 128, 64, 1024
DT = D + T


def _sc_counts(idx_flat):
    info = plsc.get_sparse_core_info()
    nc = info.num_cores

    mesh = plsc.VectorSubcoreMesh(core_axis_name="c", subcore_axis_name="s")

    @functools.partial(
        pl.kernel,
        mesh=mesh,
        compiler_params=pltpu.CompilerParams(needs_layout_passes=False),
        out_type=jax.ShapeDtypeStruct((B, SKV), jnp.float32),
        scratch_types=[
            pltpu.VMEM((K,), jnp.int32),
            pltpu.VMEM((SKV,), jnp.float32),
        ],
    )
    def counts_kernel(idx_hbm, out_hbm, idx_v, cnt_v):
        wid = lax.axis_index("s") * nc + lax.axis_index("c")
        pltpu.sync_copy(idx_hbm.at[pl.ds(wid * K, K)], idx_v)

        zeros = jnp.zeros((16,), jnp.float32)

        def zero_body(i, _):
            cnt_v[pl.ds(i * 16, 16)] = zeros
            return 0

        lax.fori_loop(0, SKV // 16, zero_body, 0)

        ones = jnp.ones((16,), jnp.float32)

        def acc_body(i, _):
            idx16 = idx_v[pl.ds(i * 16, 16)]
            plsc.addupdate_scatter(cnt_v, [idx16], ones)
            return 0

        lax.fori_loop(0, K // 16, acc_body, 0)

        pltpu.sync_copy(cnt_v, out_hbm.at[wid])

    return counts_kernel(idx_flat)


NC = 4
CS = SKV // NC


def _attn_body(q_ref, kvt_ref, cnt_ref, o_ref, m_ref, l_ref, acc_ref):
    sm_scale = 1.0 / math.sqrt(DT)
    c = pl.program_id(1)
    qb = q_ref[0]
    kvt = kvt_ref[0]
    cnt = cnt_ref[0]
    logits = lax.dot_general(
        qb, kvt, (((1,), (0,)), ((), ())),
        preferred_element_type=jnp.float32) * sm_scale
    bias = jnp.where(cnt > 0.0, jnp.log(cnt), -jnp.inf)
    logits = logits + bias
    m_cur = jnp.max(logits, axis=1, keepdims=True)

    @pl.when(c == 0)
    def _():
        m_ref[...] = jnp.full((H, 128), -jnp.inf, jnp.float32)
        l_ref[...] = jnp.zeros((H, 128), jnp.float32)
        acc_ref[...] = jnp.zeros((H, D), jnp.float32)

    m_prev = m_ref[:, :1]
    m_new = jnp.maximum(m_prev, m_cur)
    alpha = jnp.exp(m_prev - m_new)
    p = jnp.exp(logits - m_new)
    l_new = alpha * l_ref[:, :1] + jnp.sum(p, axis=1, keepdims=True)
    o_part = lax.dot_general(
        p, kvt[:D, :], (((1,), (1,)), ((), ())),
        preferred_element_type=jnp.float32)
    acc_ref[...] = alpha * acc_ref[...] + o_part
    m_ref[...] = jnp.broadcast_to(m_new, (H, 128))
    l_ref[...] = jnp.broadcast_to(l_new, (H, 128))

    @pl.when(c == NC - 1)
    def _():
        o_ref[0] = acc_ref[...] / l_ref[:, :1]


def kernel(q, kv, indices):
    idx_flat = indices.reshape(B * K)
    counts = _sc_counts(idx_flat).reshape(B, 1, SKV)

    kvt = jnp.transpose(kv, (0, 3, 2, 1)).reshape(B, DT, SKV)

    out = pl.pallas_call(
        _attn_body,
        grid=(B, NC),
        in_specs=[
            pl.BlockSpec((1, H, DT), lambda b, c: (b, 0, 0)),
            pl.BlockSpec((1, DT, CS), lambda b, c: (b, 0, c)),
            pl.BlockSpec((1, 1, CS), lambda b, c: (b, 0, c)),
        ],
        out_specs=pl.BlockSpec((1, H, D), lambda b, c: (b, 0, 0)),
        out_shape=jax.ShapeDtypeStruct((B, H, D), jnp.float32),
        scratch_shapes=[
            pltpu.VMEM((H, 128), jnp.float32),
            pltpu.VMEM((H, 128), jnp.float32),
            pltpu.VMEM((H, D), jnp.float32),
        ],
    )(q.reshape(B, H, DT), kvt, counts)
    return out.reshape(B, S, H, D)


# speedup vs baseline: 1.7257x; 1.7257x over previous
import functools
import math

import jax
import jax.numpy as jnp
from jax import lax
from jax.experimental import pallas as pl
from jax.experimental.pallas import tpu as pltpu
from jax.experimental.pallas import tpu_sc as plsc

B, S, H, SKV, G, D, T, K = 32, 1, 16, 8192, 1, 128, 64, 1024
DT = D + T


def _sc_counts(idx_flat):
    info = plsc.get_sparse_core_info()
    nc = info.num_cores

    mesh = plsc.VectorSubcoreMesh(core_axis_name="c", subcore_axis_name="s")

    @functools.partial(
        pl.kernel,
        mesh=mesh,
        compiler_params=pltpu.CompilerParams(needs_layout_passes=False),
        out_type=jax.ShapeDtypeStruct((B, 1, SKV), jnp.float32),
        scratch_types=[
            pltpu.VMEM((K,), jnp.int32),
            pltpu.VMEM((SKV,), jnp.float32),
        ],
    )
    def counts_kernel(idx_hbm, out_hbm, idx_v, cnt_v):
        wid = lax.axis_index("s") * nc + lax.axis_index("c")
        pltpu.sync_copy(idx_hbm.at[pl.ds(wid * K, K)], idx_v)

        zeros = jnp.zeros((16,), jnp.float32)

        def zero_body(i, _):
            cnt_v[pl.ds(i * 16, 16)] = zeros
            return 0

        lax.fori_loop(0, SKV // 16, zero_body, 0)

        ones = jnp.ones((16,), jnp.float32)

        def acc_body(i, _):
            idx16 = idx_v[pl.ds(i * 16, 16)]
            plsc.addupdate_scatter(cnt_v, [idx16], ones)
            return 0

        lax.fori_loop(0, K // 16, acc_body, 0)

        pltpu.sync_copy(cnt_v, out_hbm.at[wid, 0])

    return counts_kernel(idx_flat)


def _attn_body(q_ref, kvt_ref, cnt_ref, o_ref):
    sm_scale = 1.0 / math.sqrt(DT)
    qb = q_ref[0]
    kvt = kvt_ref[0]
    c = cnt_ref[0]
    logits = lax.dot_general(
        qb, kvt, (((1,), (0,)), ((), ())),
        preferred_element_type=jnp.float32) * sm_scale
    bias = jnp.where(c > 0.0, jnp.log(c), -jnp.inf)
    logits = logits + bias
    m = jnp.max(logits, axis=1, keepdims=True)
    p = jnp.exp(logits - m)
    s = jnp.sum(p, axis=1, keepdims=True)
    o = lax.dot_general(
        p, kvt[:D, :], (((1,), (1,)), ((), ())),
        preferred_element_type=jnp.float32)
    o_ref[0] = o / s


def kernel(q, kv, indices):
    idx_flat = indices.reshape(B * K)
    counts = _sc_counts(idx_flat)

    kvt = jnp.transpose(kv, (0, 3, 2, 1)).reshape(B, DT, SKV)

    out = pl.pallas_call(
        _attn_body,
        grid=(B,),
        in_specs=[
            pl.BlockSpec((1, H, DT), lambda b: (b, 0, 0)),
            pl.BlockSpec((1, DT, SKV), lambda b: (b, 0, 0)),
            pl.BlockSpec((1, 1, SKV), lambda b: (b, 0, 0)),
        ],
        out_specs=pl.BlockSpec((1, H, D), lambda b: (b, 0, 0)),
        out_shape=jax.ShapeDtypeStruct((B, H, D), jnp.float32),
    )(q.reshape(B, H, DT), kvt, counts)
    return out.reshape(B, S, H, D)
